# baseline (speedup 1.0000x reference)
import functools

import jax
import jax.numpy as jnp
from jax import lax
from jax.experimental import pallas as pl
from jax.experimental.pallas import tpu as pltpu
from jax.experimental.pallas import tpu_sc as plsc

HEIGHT = 32
WIDTH = 32
D_MODEL = 768
NUM_CORES = 2
NUM_SUBCORES = 16
CHUNK = WIDTH * D_MODEL
LANES = 16


def kernel(x, row_embed, col_embed):
    batch, seq_len, d = x.shape
    xf = x.reshape(batch * HEIGHT, CHUNK)
    colf = col_embed.reshape(1, CHUNK)
    mesh = plsc.VectorSubcoreMesh(core_axis_name="c", subcore_axis_name="s")

    @functools.partial(
        pl.kernel,
        mesh=mesh,
        out_type=jax.ShapeDtypeStruct((batch * HEIGHT, CHUNK), jnp.float32),
        scratch_types=[
            pltpu.VMEM((1, CHUNK), jnp.float32),
            pltpu.VMEM((1, D_MODEL), jnp.float32),
            pltpu.VMEM((1, CHUNK), jnp.float32),
        ],
    )
    def sc_add(x_hbm, row_hbm, col_hbm, out_hbm, pos_v, row_v, buf_v):
        h = lax.axis_index("s") * NUM_CORES + lax.axis_index("c")
        pltpu.sync_copy(col_hbm, pos_v)
        pltpu.sync_copy(row_hbm.at[pl.ds(h, 1)], row_v)

        def pos_row(r, carry):
            def pos_vec(j, carry):
                o = r * D_MODEL + j * LANES
                pos_v[0, pl.ds(o, LANES)] = (
                    pos_v[0, pl.ds(o, LANES)] + row_v[0, pl.ds(j * LANES, LANES)]
                )
                return carry

            return lax.fori_loop(0, D_MODEL // LANES, pos_vec, carry)

        lax.fori_loop(0, WIDTH, pos_row, 0)

        def per_batch(b, carry):
            r = b * HEIGHT + h
            pltpu.sync_copy(x_hbm.at[pl.ds(r, 1)], buf_v)

            def add_vec(i, carry):
                o = i * LANES
                buf_v[0, pl.ds(o, LANES)] = (
                    buf_v[0, pl.ds(o, LANES)] + pos_v[0, pl.ds(o, LANES)]
                )
                return carry

            lax.fori_loop(0, CHUNK // LANES, add_vec, carry)
            pltpu.sync_copy(buf_v, out_hbm.at[pl.ds(r, 1)])
            return carry

        lax.fori_loop(0, batch, per_batch, 0)

    out = sc_add(xf, row_embed, colf)
    return out.reshape(batch, seq_len, d)


# speedup vs baseline: 1.9183x; 1.9183x over previous
import functools

import jax
import jax.numpy as jnp
from jax import lax
from jax.experimental import pallas as pl
from jax.experimental.pallas import tpu as pltpu
from jax.experimental.pallas import tpu_sc as plsc

HEIGHT = 32
WIDTH = 32
D_MODEL = 768
NUM_CORES = 2
NUM_SUBCORES = 16
CHUNK = WIDTH * D_MODEL
LANES = 16
UNROLL = 8


def kernel(x, row_embed, col_embed):
    batch, seq_len, d = x.shape
    xf = x.reshape(batch * HEIGHT, CHUNK)
    colf = col_embed.reshape(1, CHUNK)
    mesh = plsc.VectorSubcoreMesh(core_axis_name="c", subcore_axis_name="s")

    @functools.partial(
        pl.kernel,
        mesh=mesh,
        out_type=jax.ShapeDtypeStruct((batch * HEIGHT, CHUNK), jnp.float32),
        scratch_types=[
            pltpu.VMEM((1, CHUNK), jnp.float32),
            pltpu.VMEM((1, D_MODEL), jnp.float32),
            pltpu.VMEM((1, CHUNK), jnp.float32),
            pltpu.VMEM((1, CHUNK), jnp.float32),
            pltpu.VMEM((1, CHUNK), jnp.float32),
            pltpu.VMEM((1, CHUNK), jnp.float32),
            pltpu.SemaphoreType.DMA,
            pltpu.SemaphoreType.DMA,
            pltpu.SemaphoreType.DMA,
            pltpu.SemaphoreType.DMA,
        ],
    )
    def sc_add(
        x_hbm, row_hbm, col_hbm, out_hbm,
        pos_v, row_v, in0, in1, out0, out1, isem0, isem1, osem0, osem1,
    ):
        h = lax.axis_index("s") * NUM_CORES + lax.axis_index("c")
        pltpu.sync_copy(col_hbm, pos_v)
        pltpu.sync_copy(row_hbm.at[pl.ds(h, 1)], row_v)

        def pos_row(r, carry):
            def pos_vec(j, carry):
                o = r * D_MODEL + j * LANES
                pos_v[0, pl.ds(o, LANES)] = (
                    pos_v[0, pl.ds(o, LANES)] + row_v[0, pl.ds(j * LANES, LANES)]
                )
                return carry

            return lax.fori_loop(0, D_MODEL // LANES, pos_vec, carry)

        lax.fori_loop(0, WIDTH, pos_row, 0)

        pltpu.async_copy(x_hbm.at[pl.ds(h, 1)], in0, isem0)
        pltpu.async_copy(x_hbm.at[pl.ds(HEIGHT + h, 1)], in1, isem1)

        def work(b, inb, outb, isem, osem):
            r = b * HEIGHT + h
            pltpu.make_async_copy(x_hbm.at[pl.ds(r, 1)], inb, isem).wait()

            @pl.when(b >= 2)
            def _():
                rp = (b - 2) * HEIGHT + h
                pltpu.make_async_copy(outb, out_hbm.at[pl.ds(rp, 1)], osem).wait()

            def add_vec(i, carry):
                base = i * (LANES * UNROLL)
                for u in range(UNROLL):
                    o = base + u * LANES
                    outb[0, pl.ds(o, LANES)] = (
                        inb[0, pl.ds(o, LANES)] + pos_v[0, pl.ds(o, LANES)]
                    )
                return carry

            lax.fori_loop(0, CHUNK // (LANES * UNROLL), add_vec, 0)

            @pl.when(b + 2 < batch)
            def _():
                rn = (b + 2) * HEIGHT + h
                pltpu.async_copy(x_hbm.at[pl.ds(rn, 1)], inb, isem)

            pltpu.async_copy(outb, out_hbm.at[pl.ds(r, 1)], osem)

        def per_batch(b, carry):
            even = lax.rem(b, 2) == 0

            @pl.when(even)
            def _():
                work(b, in0, out0, isem0, osem0)

            @pl.when(jnp.logical_not(even))
            def _():
                work(b, in1, out1, isem1, osem1)

            return carry

        lax.fori_loop(0, batch, per_batch, 0)

        r0 = (batch - 2) * HEIGHT + h
        r1 = (batch - 1) * HEIGHT + h
        pltpu.make_async_copy(out0, out_hbm.at[pl.ds(r0, 1)], osem0).wait()
        pltpu.make_async_copy(out1, out_hbm.at[pl.ds(r1, 1)], osem1).wait()

    out = sc_add(xf, row_embed, colf)
    return out.reshape(batch, seq_len, d)


# speedup vs baseline: 8.2193x; 4.2848x over previous
import jax
import jax.numpy as jnp
from jax.experimental import pallas as pl
from jax.experimental.pallas import tpu as pltpu

HEIGHT = 32
WIDTH = 32
D_MODEL = 768

CB = 2
NBUF = 4


def _body(x_hbm, row_ref, col_ref, out_hbm, pos, *rest):
    inbufs = rest[:NBUF]
    outbufs = rest[NBUF : 2 * NBUF]
    isems = rest[2 * NBUF : 3 * NBUF]
    osems = rest[3 * NBUF : 4 * NBUF]

    nchunks = x_hbm.shape[0] // CB

    pos[...] = row_ref[...][:, None, :] + col_ref[...][None, :, :]

    def in_copy(i, j):
        return pltpu.make_async_copy(
            x_hbm.at[pl.ds(i * CB, CB)], inbufs[j], isems[j]
        )

    def out_copy(i, j):
        return pltpu.make_async_copy(
            outbufs[j], out_hbm.at[pl.ds(i * CB, CB)], osems[j]
        )

    for j in range(NBUF):
        in_copy(j, j).start()

    for i in range(nchunks):
        j = i % NBUF
        in_copy(i, j).wait()
        if i >= NBUF:
            out_copy(i - NBUF, j).wait()
        outbufs[j][...] = inbufs[j][...] + pos[...][None]
        out_copy(i, j).start()
        if i + NBUF < nchunks:
            in_copy(i + NBUF, j).start()

    for i in range(nchunks - NBUF, nchunks):
        out_copy(i, i % NBUF).wait()


def kernel(x, row_embed, col_embed):
    batch, seq_len, d = x.shape
    x4 = x.reshape(batch, HEIGHT, WIDTH, d)
    out = pl.pallas_call(
        _body,
        in_specs=[
            pl.BlockSpec(memory_space=pl.ANY),
            pl.BlockSpec(memory_space=pltpu.VMEM),
            pl.BlockSpec(memory_space=pltpu.VMEM),
        ],
        out_specs=pl.BlockSpec(memory_space=pl.ANY),
        out_shape=jax.ShapeDtypeStruct((batch, HEIGHT, WIDTH, d), x.dtype),
        scratch_shapes=(
            [pltpu.VMEM((HEIGHT, WIDTH, d), jnp.float32)]
            + [pltpu.VMEM((CB, HEIGHT, WIDTH, d), jnp.float32) for _ in range(2 * NBUF)]
            + [pltpu.SemaphoreType.DMA for _ in range(2 * NBUF)]
        ),
        compiler_params=pltpu.CompilerParams(vmem_limit_bytes=120 * 1024 * 1024),
    )(x4, row_embed, col_embed)
    return out.reshape(batch, seq_len, d)


# speedup vs baseline: 8.2260x; 1.0008x over previous
import jax
import jax.numpy as jnp
from jax.experimental import pallas as pl
from jax.experimental.pallas import tpu as pltpu

HEIGHT = 32
WIDTH = 32
D_MODEL = 768

CB = 4
NBUF = 2


def _body(x_hbm, row_ref, col_ref, out_hbm, pos, *rest):
    inbufs = rest[:NBUF]
    outbufs = rest[NBUF : 2 * NBUF]
    isems = rest[2 * NBUF : 3 * NBUF]
    osems = rest[3 * NBUF : 4 * NBUF]

    nchunks = x_hbm.shape[0] // CB

    pos[...] = row_ref[...][:, None, :] + col_ref[...][None, :, :]

    def in_copy(i, j):
        return pltpu.make_async_copy(
            x_hbm.at[pl.ds(i * CB, CB)], inbufs[j], isems[j]
        )

    def out_copy(i, j):
        return pltpu.make_async_copy(
            outbufs[j], out_hbm.at[pl.ds(i * CB, CB)], osems[j]
        )

    for j in range(NBUF):
        in_copy(j, j).start()

    for i in range(nchunks):
        j = i % NBUF
        in_copy(i, j).wait()
        if i >= NBUF:
            out_copy(i - NBUF, j).wait()
        outbufs[j][...] = inbufs[j][...] + pos[...][None]
        out_copy(i, j).start()
        if i + NBUF < nchunks:
            in_copy(i + NBUF, j).start()

    for i in range(nchunks - NBUF, nchunks):
        out_copy(i, i % NBUF).wait()


def kernel(x, row_embed, col_embed):
    batch, seq_len, d = x.shape
    x4 = x.reshape(batch, HEIGHT, WIDTH, d)
    out = pl.pallas_call(
        _body,
        in_specs=[
            pl.BlockSpec(memory_space=pl.ANY),
            pl.BlockSpec(memory_space=pltpu.VMEM),
            pl.BlockSpec(memory_space=pltpu.VMEM),
        ],
        out_specs=pl.BlockSpec(memory_space=pl.ANY),
        out_shape=jax.ShapeDtypeStruct((batch, HEIGHT, WIDTH, d), x.dtype),
        scratch_shapes=(
            [pltpu.VMEM((HEIGHT, WIDTH, d), jnp.float32)]
            + [pltpu.VMEM((CB, HEIGHT, WIDTH, d), jnp.float32) for _ in range(2 * NBUF)]
            + [pltpu.SemaphoreType.DMA for _ in range(2 * NBUF)]
        ),
        compiler_params=pltpu.CompilerParams(vmem_limit_bytes=120 * 1024 * 1024),
    )(x4, row_embed, col_embed)
    return out.reshape(batch, seq_len, d)


# speedup vs baseline: 8.3067x; 1.0098x over previous
import jax
import jax.numpy as jnp
from jax.experimental import pallas as pl
from jax.experimental.pallas import tpu as pltpu

HEIGHT = 32
WIDTH = 32
D_MODEL = 768


B_BLK = 4


def _add_pos_body(x_ref, row_ref, col_ref, out_ref):
    pos = (row_ref[...][:, None, :] + col_ref[...][None, :, :]).reshape(
        1, HEIGHT * WIDTH, D_MODEL
    )
    out_ref[...] = x_ref[...] + pos


def kernel(x, row_embed, col_embed):
    batch, seq_len, d = x.shape
    out = pl.pallas_call(
        _add_pos_body,
        grid=(batch // B_BLK,),
        in_specs=[
            pl.BlockSpec((B_BLK, seq_len, d), lambda b: (b, 0, 0)),
            pl.BlockSpec((HEIGHT, d), lambda b: (0, 0)),
            pl.BlockSpec((WIDTH, d), lambda b: (0, 0)),
        ],
        out_specs=pl.BlockSpec((B_BLK, seq_len, d), lambda b: (b, 0, 0)),
        out_shape=jax.ShapeDtypeStruct((batch, seq_len, d), x.dtype),
        compiler_params=pltpu.CompilerParams(vmem_limit_bytes=120 * 1024 * 1024),
    )(x, row_embed, col_embed)
    return out
